# baseline (speedup 1.0000x reference)
import jax
import jax.numpy as jnp
from jax.experimental import pallas as pl


def _graph_conv(x, src, dst, Wr, Wroot, b, n):
    agg = jnp.zeros((n, x.shape[1]), x.dtype).at[dst].add(x[src])
    return agg @ Wr.T + b + x @ Wroot.T


def _inorm(x, g, bt):
    m = jnp.mean(x)
    v = jnp.mean((x - m) ** 2)
    return (x - m) / jnp.sqrt(v + 0.001) * g + bt


def _res_block(x, src, dst, convs, g3, b3, n):
    x = _graph_conv(x, src, dst, convs[0][0], convs[0][1], convs[0][2], n)
    x = _inorm(x, g3[0], b3[0])
    x1 = x
    x = _graph_conv(x, src, dst, convs[1][0], convs[1][1], convs[1][2], n)
    x = _inorm(x, g3[1], b3[1])
    x = jax.nn.relu(x)
    x = _graph_conv(x, src, dst, convs[2][0], convs[2][1], convs[2][2], n)
    x = _inorm(x, g3[2], b3[2])
    return jax.nn.relu((x + x1) / 2.0)


def _add_kernel(a_ref, b_ref, o_ref):
    o_ref[...] = a_ref[...] + b_ref[...]


def _padd(a, b):
    shp = a.shape
    a2 = a.reshape(750, 200)
    b2 = b.reshape(750, 200)
    out = pl.pallas_call(
        _add_kernel,
        out_shape=jax.ShapeDtypeStruct(a2.shape, a2.dtype),
    )(a2, b2)
    return out.reshape(shp)


def kernel(graph_features, encoder_projection, prev_results, edge_index, proc_Wr, proc_Wroot, proc_b, d0_Wr, d0_Wroot, d0_b, dW_rel, dW_root, d_b, gammas, betas, out_Wr, out_Wroot, out_b):
    src = edge_index[0]
    dst = edge_index[1]
    n = graph_features.shape[0]
    x = jax.nn.relu(_graph_conv(graph_features, src, dst, proc_Wr, proc_Wroot, proc_b, n))
    x = jnp.concatenate([x, encoder_projection, prev_results], axis=-1)
    all_convs = [(d0_Wr, d0_Wroot, d0_b)] + [(dW_rel[i], dW_root[i], d_b[i]) for i in range(8)]
    for bi in range(3):
        convs = all_convs[3 * bi:3 * bi + 3]
        x = _res_block(x, src, dst, convs, gammas[3 * bi:3 * bi + 3], betas[3 * bi:3 * bi + 3], n)
    res = _graph_conv(x, src, dst, out_Wr, out_Wroot, out_b, n)
    res = _padd(res, prev_results)
    return (x, res)


# speedup vs baseline: 10.1752x; 10.1752x over previous
import functools

import jax
import jax.numpy as jnp
from jax import lax
from jax.experimental import pallas as pl
from jax.experimental.pallas import tpu as pltpu
from jax.experimental.pallas import tpu_sc as plsc

N = 50000
F = 32
NPAD = 51200
E = 800000
EPAD = 819200
NW = 32
EPW = EPAD // NW
K = 4
GROUPS = EPW // (K * 128)
RPT = NPAD // 16
ZCH = 128


def _agg_body(y_hbm, src_hbm, dst_hbm, zero_hbm, agg_hbm,
              agg_s, idx_s, idx_d, rows, sem, zsem):
    c = lax.axis_index("c")
    s = lax.axis_index("s")
    wid = c * 16 + s

    pltpu.sync_copy(zero_hbm, rows.at[pl.ds(0, ZCH)])
    zcps = [pltpu.async_copy(rows.at[pl.ds(0, ZCH)],
                             agg_s.at[pl.ds(s * RPT + z * ZCH, ZCH)], zsem)
            for z in range(RPT // ZCH)]
    for cp in zcps:
        cp.wait()
    plsc.subcore_barrier()

    gbase = wid * (EPW // 128)

    def group(g, carry):
        rb = gbase + g * K
        pltpu.sync_copy(src_hbm.at[pl.ds(rb, K)], idx_s)
        pltpu.sync_copy(dst_hbm.at[pl.ds(rb, K)], idx_d)
        cps = [pltpu.async_copy(y_hbm.at[idx_s.at[j]],
                                rows.at[pl.ds(j * 128, 128)], sem)
               for j in range(K)]
        for cp in cps:
            cp.wait()
        for j in range(K):
            pltpu.sync_copy(rows.at[pl.ds(j * 128, 128)],
                            agg_s.at[idx_d.at[j]], add=True)
        return carry

    lax.fori_loop(0, GROUPS, group, 0)
    plsc.subcore_barrier()
    pltpu.sync_copy(agg_s.at[pl.ds(s * RPT, RPT)],
                    agg_hbm.at[c, pl.ds(s * RPT, RPT)])


_agg_call = pl.kernel(
    _agg_body,
    out_type=jax.ShapeDtypeStruct((2, NPAD, F), jnp.float32),
    mesh=plsc.VectorSubcoreMesh(core_axis_name="c", subcore_axis_name="s"),
    scratch_types=[
        pltpu.VMEM_SHARED((NPAD, F), jnp.float32),
        pltpu.VMEM((K, 128), jnp.int32),
        pltpu.VMEM((K, 128), jnp.int32),
        pltpu.VMEM((K * 128, F), jnp.float32),
        pltpu.SemaphoreType.DMA,
        pltpu.SemaphoreType.DMA,
    ],
    compiler_params=pltpu.CompilerParams(use_tc_tiling_on_sc=False),
)


def _inorm(x, g, bt):
    m = jnp.mean(x)
    v = jnp.mean((x - m) ** 2)
    return (x - m) / jnp.sqrt(v + 0.001) * g + bt


def kernel(graph_features, encoder_projection, prev_results, edge_index,
           proc_Wr, proc_Wroot, proc_b, d0_Wr, d0_Wroot, d0_b,
           dW_rel, dW_root, d_b, gammas, betas, out_Wr, out_Wroot, out_b):
    src = edge_index[0]
    dst = edge_index[1]
    pad = EPAD - E
    ar = jnp.arange(pad, dtype=jnp.int32)
    srcp = jnp.concatenate([src, (ar * 131) % N]).reshape(EPAD // 128, 128)
    dstp = jnp.concatenate([dst, N + ar % (NPAD - N)]).reshape(EPAD // 128, 128)
    zero = jnp.zeros((ZCH, F), jnp.float32)

    def aggregate(y):
        agg2 = _agg_call(y, srcp, dstp, zero)
        return agg2[0, :N] + agg2[1, :N]

    z = aggregate(graph_features) @ proc_Wr.T + proc_b \
        + graph_features @ proc_Wroot.T
    x0 = jax.nn.relu(z)
    cat = jnp.concatenate([x0, encoder_projection, prev_results], axis=-1)

    def conv32(x, Wr, Wroot, b):
        return aggregate(x) @ Wr.T + b + x @ Wroot.T

    pre = cat @ d0_Wr.T
    z = aggregate(pre) + cat @ d0_Wroot.T + d0_b
    x = _inorm(z, gammas[0], betas[0])
    x1 = x
    x = _inorm(conv32(x, dW_rel[0], dW_root[0], d_b[0]), gammas[1], betas[1])
    x = jax.nn.relu(x)
    x = _inorm(conv32(x, dW_rel[1], dW_root[1], d_b[1]), gammas[2], betas[2])
    x = jax.nn.relu((x + x1) / 2.0)

    for bi in (1, 2):
        i0 = 3 * bi - 1
        x = _inorm(conv32(x, dW_rel[i0], dW_root[i0], d_b[i0]),
                   gammas[3 * bi], betas[3 * bi])
        x1 = x
        x = _inorm(conv32(x, dW_rel[i0 + 1], dW_root[i0 + 1], d_b[i0 + 1]),
                   gammas[3 * bi + 1], betas[3 * bi + 1])
        x = jax.nn.relu(x)
        x = _inorm(conv32(x, dW_rel[i0 + 2], dW_root[i0 + 2], d_b[i0 + 2]),
                   gammas[3 * bi + 2], betas[3 * bi + 2])
        x = jax.nn.relu((x + x1) / 2.0)

    res = aggregate(x) @ out_Wr.T + out_b + x @ out_Wroot.T + prev_results
    return (x, res)


# speedup vs baseline: 12.5572x; 1.2341x over previous
import functools

import jax
import jax.numpy as jnp
from jax import lax
from jax.experimental import pallas as pl
from jax.experimental.pallas import tpu as pltpu
from jax.experimental.pallas import tpu_sc as plsc

N = 50000
F = 32
NPAD = 51200
E = 800000
GROUP = 384
NW = 32
G = 68
EPW = GROUP * G
EPAD = EPW * NW
RPT = NPAD // 16
ZCH = 128


def _agg_body(y_hbm, idx_hbm, zero_hbm, agg_hbm,
              agg_s, idx_a, idx_b, rows_a, rows_b, gsem, ssem, zsem):
    c = lax.axis_index("c")
    s = lax.axis_index("s")
    wid = c * 16 + s

    pltpu.sync_copy(zero_hbm, rows_a.at[pl.ds(0, ZCH)])
    zcps = [pltpu.async_copy(rows_a.at[pl.ds(0, ZCH)],
                             agg_s.at[pl.ds(s * RPT + z * ZCH, ZCH)], zsem)
            for z in range(RPT // ZCH)]
    for cp in zcps:
        cp.wait()
    plsc.subcore_barrier()

    gbase = wid * G

    def pair(kk, carry):
        g = gbase + 2 * kk
        pltpu.sync_copy(idx_hbm.at[g], idx_a)
        ga = pltpu.async_copy(y_hbm.at[idx_a.at[0]], rows_a, gsem)
        pltpu.sync_copy(idx_hbm.at[g + 1], idx_b)
        gb = pltpu.async_copy(y_hbm.at[idx_b.at[0]], rows_b, gsem)
        ga.wait()
        sa = pltpu.async_copy(rows_a, agg_s.at[idx_a.at[1]], ssem, add=True)
        gb.wait()
        sb = pltpu.async_copy(rows_b, agg_s.at[idx_b.at[1]], ssem, add=True)
        sa.wait()
        sb.wait()
        return carry

    lax.fori_loop(0, G // 2, pair, 0)
    plsc.subcore_barrier()
    pltpu.sync_copy(agg_s.at[pl.ds(s * RPT, RPT)],
                    agg_hbm.at[c, pl.ds(s * RPT, RPT)])


_agg_call = pl.kernel(
    _agg_body,
    out_type=jax.ShapeDtypeStruct((2, NPAD, F), jnp.float32),
    mesh=plsc.VectorSubcoreMesh(core_axis_name="c", subcore_axis_name="s"),
    scratch_types=[
        pltpu.VMEM_SHARED((NPAD, F), jnp.float32),
        pltpu.VMEM((2, GROUP), jnp.int32),
        pltpu.VMEM((2, GROUP), jnp.int32),
        pltpu.VMEM((GROUP, F), jnp.float32),
        pltpu.VMEM((GROUP, F), jnp.float32),
        pltpu.SemaphoreType.DMA,
        pltpu.SemaphoreType.DMA,
        pltpu.SemaphoreType.DMA,
    ],
    compiler_params=pltpu.CompilerParams(use_tc_tiling_on_sc=False),
)


def _inorm(x, g, bt):
    m = jnp.mean(x)
    v = jnp.mean((x - m) ** 2)
    return (x - m) / jnp.sqrt(v + 0.001) * g + bt


def kernel(graph_features, encoder_projection, prev_results, edge_index,
           proc_Wr, proc_Wroot, proc_b, d0_Wr, d0_Wroot, d0_b,
           dW_rel, dW_root, d_b, gammas, betas, out_Wr, out_Wroot, out_b):
    src = edge_index[0]
    dst = edge_index[1]
    pad = EPAD - E
    ar = jnp.arange(pad, dtype=jnp.int32)
    srcp = jnp.concatenate([src, (ar * 131) % N]).reshape(-1, 1, GROUP)
    dstp = jnp.concatenate([dst, N + ar % (NPAD - N)]).reshape(-1, 1, GROUP)
    idx_all = jnp.concatenate([srcp, dstp], axis=1)
    zero = jnp.zeros((ZCH, F), jnp.float32)

    def aggregate(y):
        agg2 = _agg_call(y, idx_all, zero)
        return agg2[0, :N] + agg2[1, :N]

    z = aggregate(graph_features) @ proc_Wr.T + proc_b \
        + graph_features @ proc_Wroot.T
    x0 = jax.nn.relu(z)
    cat = jnp.concatenate([x0, encoder_projection, prev_results], axis=-1)

    def conv32(x, Wr, Wroot, b):
        return aggregate(x) @ Wr.T + b + x @ Wroot.T

    pre = cat @ d0_Wr.T
    z = aggregate(pre) + cat @ d0_Wroot.T + d0_b
    x = _inorm(z, gammas[0], betas[0])
    x1 = x
    x = _inorm(conv32(x, dW_rel[0], dW_root[0], d_b[0]), gammas[1], betas[1])
    x = jax.nn.relu(x)
    x = _inorm(conv32(x, dW_rel[1], dW_root[1], d_b[1]), gammas[2], betas[2])
    x = jax.nn.relu((x + x1) / 2.0)

    for bi in (1, 2):
        i0 = 3 * bi - 1
        x = _inorm(conv32(x, dW_rel[i0], dW_root[i0], d_b[i0]),
                   gammas[3 * bi], betas[3 * bi])
        x1 = x
        x = _inorm(conv32(x, dW_rel[i0 + 1], dW_root[i0 + 1], d_b[i0 + 1]),
                   gammas[3 * bi + 1], betas[3 * bi + 1])
        x = jax.nn.relu(x)
        x = _inorm(conv32(x, dW_rel[i0 + 2], dW_root[i0 + 2], d_b[i0 + 2]),
                   gammas[3 * bi + 2], betas[3 * bi + 2])
        x = jax.nn.relu((x + x1) / 2.0)

    res = aggregate(x) @ out_Wr.T + out_b + x @ out_Wroot.T + prev_results
    return (x, res)
